# Initial kernel scaffold; baseline (speedup 1.0000x reference)
import jax
import jax.numpy as jnp
from jax.experimental import pallas as pl


def kernel(x, rows_ih, cols_ih, values_ih, rows_hh, cols_hh, values_hh, bias_ih, bias_hh):
    raise NotImplementedError("write your pallas kernel here")


# speedup vs baseline: 4.3784x; 4.3784x over previous
import functools

import jax
import jax.numpy as jnp
from jax import lax
from jax.experimental import pallas as pl
from jax.experimental.pallas import tpu as pltpu
from jax.experimental.pallas import tpu_sc as plsc

H = 16384
B = 64
T = 16
NC = 2
NS = 16
BH = B // NC
CK = 128
ROWS_PER_TILE = H // NS
RCH = ROWS_PER_TILE // CK


def _tanh16(y):
    e = jnp.exp(y * 2.0)
    return 1.0 - 2.0 / (e + 1.0)


def _sc_rnn(nch, xt, cols_ih, rows_ih, vals_ih, cols_hh, rows_hh, vals_hh,
            bias_ih, bias_hh):
    mesh = plsc.VectorSubcoreMesh(core_axis_name="c", subcore_axis_name="s")

    @functools.partial(
        pl.kernel,
        mesh=mesh,
        compiler_params=pltpu.CompilerParams(use_tc_tiling_on_sc=False),
        out_type=jax.ShapeDtypeStruct((T, NC, H, BH), jnp.float32),
        scratch_types=[
            pltpu.VMEM((CK,), jnp.int32),
            pltpu.VMEM((CK,), jnp.int32),
            pltpu.VMEM((CK,), jnp.int32),
            pltpu.VMEM((CK,), jnp.float32),
            pltpu.VMEM((CK, BH), jnp.float32),
            pltpu.VMEM((CK, BH), jnp.float32),
            pltpu.VMEM((CK, BH), jnp.float32),
            pltpu.VMEM((CK,), jnp.float32),
            pltpu.VMEM((CK,), jnp.float32),
            pltpu.VMEM_SHARED((H, BH), jnp.float32),
            pltpu.VMEM_SHARED((H, BH), jnp.float32),
            pltpu.SemaphoreType.DMA,
        ],
    )
    def rnn(xt_hbm, ci_hbm, ri_hbm, vi_hbm, ch_hbm, rh_hbm, vh_hbm,
            bi_hbm, bh_hbm, out_hbm,
            idx_raw, idx_ih, row_buf, val_buf, gbuf, tbuf, zbuf, bbi, bbh,
            h_sh, acc_sh, sem):
        cid = lax.axis_index("c")
        sid = lax.axis_index("s")
        row0 = sid * ROWS_PER_TILE
        iota = lax.broadcasted_iota(jnp.int32, (16,), 0)

        def zrow(k, _):
            z = jnp.zeros((16,), jnp.float32)
            zbuf[k, pl.ds(0, 16)] = z
            zbuf[k, pl.ds(16, 16)] = z
            return 0
        lax.fori_loop(0, CK, zrow, 0)

        def zchunk(r, _):
            base = row0 + r * CK
            pltpu.sync_copy(zbuf, h_sh.at[pl.ds(base, CK)])
            pltpu.sync_copy(zbuf, acc_sh.at[pl.ds(base, CK)])
            return 0
        lax.fori_loop(0, RCH, zchunk, 0)
        plsc.subcore_barrier()

        def scale_chunk(buf):
            def grp(g, _):
                vg = val_buf[pl.ds(g * 16, 16)]
                for j in range(16):
                    k = g * 16 + j
                    vv = vg[j]
                    for off in (0, 16):
                        s = pl.ds(off, 16)
                        buf[k, s] = buf[k, s] * vv
                return 0
            lax.fori_loop(0, CK // 16, grp, 0)

        def step(t, _):
            xoff = cid * (T * H) + t * H

            def ih_chunk(i, _):
                pltpu.sync_copy(ci_hbm.at[sid, i], idx_raw)
                pltpu.sync_copy(ri_hbm.at[sid, i], row_buf)
                pltpu.sync_copy(vi_hbm.at[sid, i], val_buf)
                def addoff(g, _):
                    s = pl.ds(g * 16, 16)
                    idx_ih[s] = idx_raw[s] + xoff
                    return 0
                lax.fori_loop(0, CK // 16, addoff, 0)
                pltpu.async_copy(xt_hbm.at[idx_ih], gbuf, sem).wait()
                scale_chunk(gbuf)
                pltpu.sync_copy(gbuf, acc_sh.at[row_buf], add=True)
                return 0
            lax.fori_loop(0, nch, ih_chunk, 0)

            def hh_chunk(i, _):
                pltpu.sync_copy(ch_hbm.at[sid, i], idx_raw)
                pltpu.sync_copy(rh_hbm.at[sid, i], row_buf)
                pltpu.sync_copy(vh_hbm.at[sid, i], val_buf)
                pltpu.async_copy(h_sh.at[idx_raw], gbuf, sem).wait()
                scale_chunk(gbuf)
                pltpu.sync_copy(gbuf, acc_sh.at[row_buf], add=True)
                return 0
            lax.fori_loop(0, nch, hh_chunk, 0)

            plsc.subcore_barrier()

            def trow_chunk(r, _):
                base = row0 + r * CK
                pltpu.sync_copy(acc_sh.at[pl.ds(base, CK)], tbuf)
                pltpu.sync_copy(zbuf, acc_sh.at[pl.ds(base, CK)])
                pltpu.sync_copy(bi_hbm.at[pl.ds(base, CK)], bbi)
                pltpu.sync_copy(bh_hbm.at[pl.ds(base, CK)], bbh)
                def badd(g, _):
                    s = pl.ds(g * 16, 16)
                    bbi[s] = bbi[s] + bbh[s]
                    return 0
                lax.fori_loop(0, CK // 16, badd, 0)
                def grp(g, _):
                    bg = bbi[pl.ds(g * 16, 16)]
                    for j in range(16):
                        k = g * 16 + j
                        bv = bg[j]
                        for off in (0, 16):
                            s = pl.ds(off, 16)
                            tbuf[k, s] = _tanh16(tbuf[k, s] + bv)
                    return 0
                lax.fori_loop(0, CK // 16, grp, 0)
                pltpu.sync_copy(tbuf, h_sh.at[pl.ds(base, CK)])
                pltpu.sync_copy(tbuf, out_hbm.at[t, cid, pl.ds(base, CK)])
                return 0
            lax.fori_loop(0, RCH, trow_chunk, 0)

            plsc.subcore_barrier()
            return 0

        lax.fori_loop(0, T, step, 0)

    return rnn(xt, cols_ih, rows_ih, vals_ih, cols_hh, rows_hh, vals_hh,
               bias_ih, bias_hh)


def _prep_edges(rows, cols, vals, npad):
    nnz = rows.shape[0]
    padn = npad - nnz
    fill = (jnp.arange(padn, dtype=jnp.int32) * 16) % H
    r = jnp.concatenate([rows, fill])
    c = jnp.concatenate([cols, fill])
    v = jnp.concatenate([vals, jnp.zeros((padn,), vals.dtype)])
    nch = npad // (NS * CK)
    return (r.reshape(NS, nch, CK), c.reshape(NS, nch, CK),
            v.reshape(NS, nch, CK), nch)


def kernel(x, rows_ih, cols_ih, values_ih, rows_hh, cols_hh, values_hh,
           bias_ih, bias_hh):
    nnz = rows_ih.shape[0]
    npad = ((nnz + NS * CK - 1) // (NS * CK)) * (NS * CK)
    ri, ci, vi, nch = _prep_edges(rows_ih, cols_ih, values_ih, npad)
    rh, ch, vh, _ = _prep_edges(rows_hh, cols_hh, values_hh, npad)
    xt = jnp.transpose(x, (1, 2, 0)).reshape(T, H, NC, BH)
    xt = jnp.transpose(xt, (2, 0, 1, 3)).reshape(NC * T * H, BH)
    bi = bias_ih.reshape(H)
    bh = bias_hh.reshape(H)
    out = _sc_rnn(nch, xt, ci, ri, vi, ch, rh, vh, bi, bh)
    return jnp.transpose(out, (1, 3, 0, 2)).reshape(B, T, H)
